# baseline (speedup 1.0000x reference)
import functools

import jax
import jax.numpy as jnp
from jax import lax
from jax.experimental import pallas as pl
from jax.experimental.pallas import tpu as pltpu
from jax.experimental.pallas import tpu_sc as plsc

NMERS = 6
HID = 768
LAB = 16
B = 16
S = 512
P = S + NMERS - 1

NC = 2
NS = 16
CHUNK = 264
PPAD = 2 * CHUNK


def _proj_kernel(emb_ref, wt_ref, out_ref):
    out_ref[0] = jnp.dot(emb_ref[0], wt_ref[:],
                         preferred_element_type=jnp.float32)


def _sc_unfold(proj_hbm, mask_hbm, b_hbm, out_hbm,
               prow_v, mrow_v, obuf_v, bvec_v):
    c = lax.axis_index("c")
    s = lax.axis_index("s")
    wid = s * NC + c
    row = wid // 2
    half = wid % 2
    qlo = half * CHUNK

    pltpu.sync_copy(proj_hbm.at[row], prow_v)
    pltpu.sync_copy(mask_hbm.at[row], mrow_v)
    pltpu.sync_copy(b_hbm, bvec_v)
    bvec = bvec_v[...]

    def _msum(k, acc):
        return acc + mrow_v[pl.ds(k * LAB, LAB)]

    macc = lax.fori_loop(0, S // LAB, _msum, jnp.zeros((LAB,), jnp.int32))
    L = macc[0]
    for k in range(1, LAB):
        L = L + macc[k]
    nc = jnp.maximum(L - 2, 1)

    def pm(i):
        ii = jnp.clip(i, 0, S - 1)
        f = jnp.where((i >= 1) & (i <= nc), 1.0, 0.0).astype(jnp.float32)
        return prow_v[ii] * f

    def _init(i, acc):
        return acc + pm(i)

    acc0 = lax.fori_loop(qlo - 5, qlo, _init, jnp.zeros((LAB,), jnp.float32))

    def _body(j, acc):
        q = qlo + j
        acc = acc + pm(q)
        lo = jnp.maximum(1, q - 5)
        hi = jnp.minimum(nc, q)
        den = jnp.maximum(hi - lo + 1, 1).astype(jnp.float32)
        obuf_v[j] = acc / den + bvec
        return acc - pm(q - 5)

    lax.fori_loop(0, CHUNK, _body, acc0)

    @pl.when(half == 0)
    def _():
        obuf_v[0] = prow_v[0] + bvec

    eidx = jnp.where(L >= 1, L - 1, S - 1)
    qe = L + 4

    @pl.when((qe >= qlo) & (qe < qlo + CHUNK))
    def _():
        obuf_v[qe - qlo] = prow_v[eidx] + bvec

    pltpu.sync_copy(obuf_v, out_hbm.at[row, pl.ds(qlo, CHUNK)])


@jax.jit
def kernel(outputs, attention_mask, W_dec, b_dec):
    emb = outputs[0]
    wt = W_dec.T

    proj = pl.pallas_call(
        _proj_kernel,
        grid=(B,),
        in_specs=[
            pl.BlockSpec((1, S, HID), lambda b: (b, 0, 0)),
            pl.BlockSpec((HID, LAB), lambda b: (0, 0)),
        ],
        out_specs=pl.BlockSpec((1, S, LAB), lambda b: (b, 0, 0)),
        out_shape=jax.ShapeDtypeStruct((B, S, LAB), jnp.float32),
    )(emb, wt)

    sc_unfold = pl.kernel(
        _sc_unfold,
        out_type=jax.ShapeDtypeStruct((B, PPAD, LAB), jnp.float32),
        mesh=plsc.VectorSubcoreMesh(
            core_axis_name="c", subcore_axis_name="s",
            num_cores=NC, num_subcores=NS),
        scratch_types=[
            pltpu.VMEM((S, LAB), jnp.float32),
            pltpu.VMEM((S,), jnp.int32),
            pltpu.VMEM((CHUNK, LAB), jnp.float32),
            pltpu.VMEM((LAB,), jnp.float32),
        ],
    )

    out = sc_unfold(proj, attention_mask, b_dec)
    return out[:, :P, :]


# speedup vs baseline: 1.0020x; 1.0020x over previous
import functools

import jax
import jax.numpy as jnp
from jax import lax
from jax.experimental import pallas as pl
from jax.experimental.pallas import tpu as pltpu
from jax.experimental.pallas import tpu_sc as plsc

NMERS = 6
HID = 768
LAB = 16
B = 16
S = 512
P = S + NMERS - 1

NC = 2
NS = 16
CHUNK = 264
PPAD = 2 * CHUNK


def _proj_kernel(emb_ref, wt_ref, out_ref):
    out_ref[0] = jnp.dot(emb_ref[0], wt_ref[:],
                         preferred_element_type=jnp.float32)


def _sc_unfold(proj_hbm, mask_hbm, b_hbm, out_hbm,
               prow_v, mrow_v, obuf_v, bvec_v):
    c = lax.axis_index("c")
    s = lax.axis_index("s")
    wid = s * NC + c
    row = wid // 2
    half = wid % 2
    qlo = half * CHUNK

    pltpu.sync_copy(proj_hbm.at[row], prow_v)
    pltpu.sync_copy(mask_hbm.at[row], mrow_v)
    pltpu.sync_copy(b_hbm, bvec_v)
    bvec = bvec_v[...]

    def _msum(k, acc):
        return acc + mrow_v[pl.ds(k * LAB, LAB)]

    macc = lax.fori_loop(0, S // LAB, _msum, jnp.zeros((LAB,), jnp.int32),
                         unroll=8)
    L = macc[0]
    for k in range(1, LAB):
        L = L + macc[k]
    nc = jnp.maximum(L - 2, 1)

    def pm(i):
        ii = jnp.clip(i, 0, S - 1)
        f = jnp.where((i >= 1) & (i <= nc), 1.0, 0.0).astype(jnp.float32)
        return prow_v[ii] * f

    acc0 = jnp.zeros((LAB,), jnp.float32)
    for k in range(5):
        acc0 = acc0 + pm(qlo - 5 + k)

    def _body(j, acc):
        q = qlo + j
        acc = acc + pm(q)
        lo = jnp.maximum(1, q - 5)
        hi = jnp.minimum(nc, q)
        den = jnp.maximum(hi - lo + 1, 1)
        rcp = jnp.float32(1.0)
        for d in range(2, NMERS + 1):
            rcp = jnp.where(den == d, jnp.float32(1.0 / d), rcp)
        obuf_v[j] = acc * rcp + bvec
        return acc - pm(q - 5)

    lax.fori_loop(0, CHUNK, _body, acc0, unroll=8)

    @pl.when(half == 0)
    def _():
        obuf_v[0] = prow_v[0] + bvec

    eidx = jnp.where(L >= 1, L - 1, S - 1)
    qe = L + 4

    @pl.when((qe >= qlo) & (qe < qlo + CHUNK))
    def _():
        obuf_v[qe - qlo] = prow_v[eidx] + bvec

    pltpu.sync_copy(obuf_v, out_hbm.at[row, pl.ds(qlo, CHUNK)])


@jax.jit
def kernel(outputs, attention_mask, W_dec, b_dec):
    emb = outputs[0]
    wt = W_dec.T

    proj = pl.pallas_call(
        _proj_kernel,
        grid=(B,),
        in_specs=[
            pl.BlockSpec((1, S, HID), lambda b: (b, 0, 0)),
            pl.BlockSpec((HID, LAB), lambda b: (0, 0)),
        ],
        out_specs=pl.BlockSpec((1, S, LAB), lambda b: (b, 0, 0)),
        out_shape=jax.ShapeDtypeStruct((B, S, LAB), jnp.float32),
    )(emb, wt)

    sc_unfold = pl.kernel(
        _sc_unfold,
        out_type=jax.ShapeDtypeStruct((B, PPAD, LAB), jnp.float32),
        mesh=plsc.VectorSubcoreMesh(
            core_axis_name="c", subcore_axis_name="s",
            num_cores=NC, num_subcores=NS),
        scratch_types=[
            pltpu.VMEM((S, LAB), jnp.float32),
            pltpu.VMEM((S,), jnp.int32),
            pltpu.VMEM((CHUNK, LAB), jnp.float32),
            pltpu.VMEM((LAB,), jnp.float32),
        ],
    )

    out = sc_unfold(proj, attention_mask, b_dec)
    return out[:, :P, :]
